# baseline (speedup 1.0000x reference)
import jax
import jax.numpy as jnp
from jax import lax
from jax.experimental import pallas as pl
from jax.experimental.pallas import tpu as pltpu
from jax.experimental.pallas import tpu_sc as plsc


def _vgather(vec, idx16):
    return lax.gather(
        vec, idx16.reshape(16, 1),
        lax.GatherDimensionNumbers(offset_dims=(), collapsed_slice_dims=(0,),
                                   start_index_map=(0,)),
        (1,), mode=lax.GatherScatterMode.PROMISE_IN_BOUNDS)


def _lrelu_exp(t):
    return jnp.exp(jnp.where(t >= 0, t, 0.2 * t))


_NC = 2
_NS = 16
_NW = _NC * _NS
_CHUNK = 128
_ROWBLK = 1024


def _dense1_body(x_ref, w1_ref, asrc_ref, adst_ref, t1_ref, ad_ref):
    h = jnp.dot(x_ref[...], w1_ref[...], preferred_element_type=jnp.float32)
    b = h.shape[0]
    hh, cc = asrc_ref.shape
    hr = h.reshape(b, hh, cc)
    a_s = (hr * asrc_ref[...][None]).sum(-1)
    a_d = (hr * adst_ref[...][None]).sum(-1)
    z = jnp.zeros((b, 8), jnp.float32)
    t1_ref[...] = jnp.concatenate([h, a_s], axis=1)
    ad_ref[...] = jnp.concatenate([a_d, z], axis=1)


def _combine1_body(pa_ref, pb_ref, t1_ref, ad_ref, b1_ref, w2_ref, as2_ref,
                   ad2_ref, tab2_ref):
    acc = pa_ref[...] + pb_ref[...]
    b = acc.shape[0]
    h1 = t1_ref[:, :64]
    t = t1_ref[:, 64:72] + ad_ref[:, :8]
    wself = _lrelu_exp(t)
    num = acc[:, :64].reshape(b, 8, 8) + wself[:, :, None] * h1.reshape(b, 8, 8)
    den = acc[:, 64:72] + wself
    out1 = (num / (den[:, :, None] + 1e-16)).reshape(b, 64) + b1_ref[...]
    g = jnp.where(out1 > 0, out1, jnp.exp(jnp.minimum(out1, 0.0)) - 1.0)
    h2 = jnp.dot(g, w2_ref[...], preferred_element_type=jnp.float32)
    h2r = h2.reshape(b, as2_ref.shape[0], as2_ref.shape[1])
    s2 = (h2r * as2_ref[...][None]).sum(-1)
    d2 = (h2r * ad2_ref[...][None]).sum(-1)
    z12 = jnp.zeros((b, 12), jnp.float32)
    tab2_ref[...] = jnp.concatenate([h2, s2, d2, z12], axis=1)


def _final_body(pa_ref, pb_ref, tab2_ref, b2_ref, out_ref):
    acc = pa_ref[...] + pb_ref[...]
    h2 = tab2_ref[:, :2]
    t = tab2_ref[:, 2:3] + tab2_ref[:, 3:4]
    wself = _lrelu_exp(t)
    num = acc[:, :2] + wself * h2
    den = acc[:, 2:3] + wself
    o = num / (den + 1e-16) + b2_ref[...]
    m = jnp.max(o, axis=1, keepdims=True)
    out_ref[...] = (o - m) - jnp.log(jnp.exp(o - m).sum(axis=1, keepdims=True))


def _make_edge1_body(nch_a, nch_b):
  def _edge1_body(t1, ad1, srcg, dstg, zer, out, srcv, dstv,
                  trows0, trows1, adrows0, adrows1, sbuf0, sbuf1,
                  gt0, gt1, ga0, ga1, ss0, ss1, acc):
    cid = lax.axis_index("c")
    sid = lax.axis_index("s")
    wid = sid * _NC + cid
    if nch_a == nch_b:
        nch_w, npair = nch_a, nch_a // 2
    else:
        nch_w = jnp.where(cid == 0, nch_a, nch_b)
        npair = jnp.where(cid == 0, nch_a // 2, nch_b // 2)
    rows_per = acc.shape[0] // _NS
    trows = [trows0, trows1]
    adrows = [adrows0, adrows1]
    sbuf = [sbuf0, sbuf1]
    gt = [gt0, gt1]
    ga = [ga0, ga1]
    ss = [ss0, ss1]

    pltpu.sync_copy(srcg.at[wid], srcv)
    pltpu.sync_copy(dstg.at[wid], dstv)
    pltpu.sync_copy(zer.at[pl.ds(sid * rows_per, rows_per)],
                    acc.at[pl.ds(sid * rows_per, rows_per)])
    plsc.subcore_barrier()

    lane = lax.iota(jnp.int32, 16)
    lanem7 = lane & 7
    hi8 = lane >> 3
    widx = [8 + hi8 + 2 * jj for jj in range(4)]

    for b in range(2):
        pltpu.async_copy(t1.at[srcv.at[b]], trows[b], gt[b])
        pltpu.async_copy(ad1.at[dstv.at[b]], adrows[b], ga[b])

    def _pair(jo, carry):
        for b in range(2):
            j = 2 * jo + b

            @pl.when(jo >= 1)
            def _():
                pltpu.make_async_copy(sbuf[b], acc.at[dstv.at[j]], ss[b]).wait()

            pltpu.make_async_copy(t1.at[srcv.at[j]], trows[b], gt[b]).wait()
            pltpu.make_async_copy(ad1.at[dstv.at[j]], adrows[b], ga[b]).wait()

            @plsc.parallel_loop(0, _CHUNK, step=1, unroll=4)
            def _edge(e):
                as_v = trows[b][e, pl.ds(56, 16)]
                ad_v = adrows[b][e, pl.ds(0, 16)]
                adx = _vgather(ad_v, lanem7)
                w = _lrelu_exp(as_v + adx)
                sbuf[b][e, pl.ds(56, 16)] = w
                for jj in range(4):
                    wexp = _vgather(w, widx[jj])
                    sbuf[b][e, pl.ds(jj * 16, 16)] = (
                        trows[b][e, pl.ds(jj * 16, 16)] * wexp)

            pltpu.async_copy(sbuf[b], acc.at[dstv.at[j]], ss[b], add=True)

            @pl.when(jo < npair - 1)
            def _():
                pltpu.async_copy(t1.at[srcv.at[j + 2]], trows[b], gt[b])
                pltpu.async_copy(ad1.at[dstv.at[j + 2]], adrows[b], ga[b])
        return carry
    lax.fori_loop(0, npair, _pair, 0)

    for b in range(2):
        pltpu.make_async_copy(sbuf[b], acc.at[dstv.at[nch_w - 2 + b]],
                              ss[b]).wait()

    plsc.subcore_barrier()
    pltpu.sync_copy(acc.at[pl.ds(sid * rows_per, rows_per)],
                    out.at[cid].at[pl.ds(sid * rows_per, rows_per)])
  return _edge1_body


def _make_edge2_body(nch_a, nch_b):
  def _edge2_body(tab2h, srcg, dstg, zer, out, srcv, dstv,
                  srows0, srows1, drows0, drows1, s20, s21,
                  gt0, gt1, ga0, ga1, ss0, ss1, acc):
    cid = lax.axis_index("c")
    sid = lax.axis_index("s")
    wid = sid * _NC + cid
    if nch_a == nch_b:
        nch_w, npair = nch_a, nch_a // 2
    else:
        nch_w = jnp.where(cid == 0, nch_a, nch_b)
        npair = jnp.where(cid == 0, nch_a // 2, nch_b // 2)
    rows_per = acc.shape[0] // _NS
    srows = [srows0, srows1]
    drows = [drows0, drows1]
    s2 = [s20, s21]
    gt = [gt0, gt1]
    ga = [ga0, ga1]
    ss = [ss0, ss1]

    pltpu.sync_copy(srcg.at[wid], srcv)
    pltpu.sync_copy(dstg.at[wid], dstv)
    pltpu.sync_copy(zer.at[pl.ds(sid * rows_per, rows_per)],
                    acc.at[pl.ds(sid * rows_per, rows_per)])
    plsc.subcore_barrier()

    lane = lax.iota(jnp.int32, 16)
    l0 = lane == 0
    l1 = lane == 1
    l2 = lane == 2
    f0 = lane * 0
    f1 = f0 + 1
    f2 = f0 + 2
    f3 = f0 + 3

    for b in range(2):
        pltpu.async_copy(tab2h.at[srcv.at[b]], srows[b], gt[b])
        pltpu.async_copy(tab2h.at[dstv.at[b]], drows[b], ga[b])

    def _pair(jo, carry):
        for b in range(2):
            j = 2 * jo + b

            @pl.when(jo >= 1)
            def _():
                pltpu.make_async_copy(s2[b], acc.at[dstv.at[j]], ss[b]).wait()

            pltpu.make_async_copy(tab2h.at[srcv.at[j]], srows[b], gt[b]).wait()
            pltpu.make_async_copy(tab2h.at[dstv.at[j]], drows[b], ga[b]).wait()

            @plsc.parallel_loop(0, _CHUNK, step=1, unroll=4)
            def _edge(e):
                sr = srows[b][e, pl.ds(0, 16)]
                dr = drows[b][e, pl.ds(0, 16)]
                t = _vgather(sr, f2) + _vgather(dr, f3)
                w = _lrelu_exp(t)
                h0 = _vgather(sr, f0)
                h1 = _vgather(sr, f1)
                s2[b][e, pl.ds(0, 16)] = w * jnp.where(
                    l0, h0, jnp.where(l1, h1, jnp.where(l2, 1.0, 0.0)))

            pltpu.async_copy(s2[b], acc.at[dstv.at[j]], ss[b], add=True)

            @pl.when(jo < npair - 1)
            def _():
                pltpu.async_copy(tab2h.at[srcv.at[j + 2]], srows[b], gt[b])
                pltpu.async_copy(tab2h.at[dstv.at[j + 2]], drows[b], ga[b])
        return carry
    lax.fori_loop(0, npair, _pair, 0)

    for b in range(2):
        pltpu.make_async_copy(s2[b], acc.at[dstv.at[nch_w - 2 + b]],
                              ss[b]).wait()

    plsc.subcore_barrier()
    pltpu.sync_copy(acc.at[pl.ds(sid * rows_per, rows_per)],
                    out.at[cid].at[pl.ds(sid * rows_per, rows_per)])
  return _edge2_body


def kernel(x, edge_index, W1, att_src1, att_dst1, b1, W2, att_src2, att_dst2,
           b2):
    n, d = x.shape
    e = edge_index.shape[1]
    np_ = ((n + _ROWBLK - 1) // _ROWBLK) * _ROWBLK
    nch = (e + _NW * _CHUNK - 1) // (_NW * _CHUNK)
    nch = ((nch + 1) // 2) * 2

    nch_a = nch_b = nchm = nch
    epad = _NW * nch * _CHUNK
    src = jnp.concatenate(
        [edge_index[0], jnp.full((epad - e,), n, jnp.int32)]).reshape(
            _NW, nch, _CHUNK)
    dst = jnp.concatenate(
        [edge_index[1], jnp.full((epad - e,), n, jnp.int32)]).reshape(
            _NW, nch, _CHUNK)
    xp = jnp.pad(x, ((0, np_ - n), (0, 0)))
    zeros72 = jnp.zeros((np_, 72), jnp.float32)
    zeros16 = jnp.zeros((np_, 16), jnp.float32)

    grid = np_ // _ROWBLK
    full = lambda *shape: pl.BlockSpec(shape, lambda i: (0,) * len(shape))
    rowblk = lambda *rest: pl.BlockSpec((_ROWBLK,) + rest,
                                        lambda i: (i,) + (0,) * len(rest))

    t1, ad1 = pl.pallas_call(
        _dense1_body,
        grid=(grid,),
        in_specs=[rowblk(d), full(d, 64), full(8, 8), full(8, 8)],
        out_specs=[rowblk(72), rowblk(16)],
        out_shape=[jax.ShapeDtypeStruct((np_, 72), jnp.float32),
                   jax.ShapeDtypeStruct((np_, 16), jnp.float32)],
    )(xp, W1, att_src1, att_dst1)

    mesh = plsc.VectorSubcoreMesh(core_axis_name="c", subcore_axis_name="s")
    scparams = pltpu.CompilerParams(use_tc_tiling_on_sc=False)
    p1 = pl.kernel(
        _make_edge1_body(nch_a, nch_b),
        out_type=jax.ShapeDtypeStruct((_NC, np_, 72), jnp.float32),
        mesh=mesh,
        compiler_params=scparams,
        scratch_types=[
            pltpu.VMEM((nchm, _CHUNK), jnp.int32),
            pltpu.VMEM((nchm, _CHUNK), jnp.int32),
            pltpu.VMEM((_CHUNK, 72), jnp.float32),
            pltpu.VMEM((_CHUNK, 72), jnp.float32),
            pltpu.VMEM((_CHUNK, 16), jnp.float32),
            pltpu.VMEM((_CHUNK, 16), jnp.float32),
            pltpu.VMEM((_CHUNK, 72), jnp.float32),
            pltpu.VMEM((_CHUNK, 72), jnp.float32),
            pltpu.SemaphoreType.DMA,
            pltpu.SemaphoreType.DMA,
            pltpu.SemaphoreType.DMA,
            pltpu.SemaphoreType.DMA,
            pltpu.SemaphoreType.DMA,
            pltpu.SemaphoreType.DMA,
            pltpu.VMEM_SHARED((np_, 72), jnp.float32),
        ],
    )(t1, ad1, src, dst, zeros72)

    tab2 = pl.pallas_call(
        _combine1_body,
        grid=(grid,),
        in_specs=[rowblk(72), rowblk(72), rowblk(72), rowblk(16), full(1, 64),
                  full(64, 2), full(1, 2), full(1, 2)],
        out_specs=[rowblk(16)],
        out_shape=[jax.ShapeDtypeStruct((np_, 16), jnp.float32)],
    )(p1[0], p1[1], t1, ad1, b1.reshape(1, 64), W2, att_src2, att_dst2)[0]

    p2 = pl.kernel(
        _make_edge2_body(nch_a, nch_b),
        out_type=jax.ShapeDtypeStruct((_NC, np_, 16), jnp.float32),
        mesh=mesh,
        compiler_params=scparams,
        scratch_types=[
            pltpu.VMEM((nchm, _CHUNK), jnp.int32),
            pltpu.VMEM((nchm, _CHUNK), jnp.int32),
            pltpu.VMEM((_CHUNK, 16), jnp.float32),
            pltpu.VMEM((_CHUNK, 16), jnp.float32),
            pltpu.VMEM((_CHUNK, 16), jnp.float32),
            pltpu.VMEM((_CHUNK, 16), jnp.float32),
            pltpu.VMEM((_CHUNK, 16), jnp.float32),
            pltpu.VMEM((_CHUNK, 16), jnp.float32),
            pltpu.SemaphoreType.DMA,
            pltpu.SemaphoreType.DMA,
            pltpu.SemaphoreType.DMA,
            pltpu.SemaphoreType.DMA,
            pltpu.SemaphoreType.DMA,
            pltpu.SemaphoreType.DMA,
            pltpu.VMEM_SHARED((np_, 16), jnp.float32),
        ],
    )(tab2, src, dst, zeros16)

    outp = pl.pallas_call(
        _final_body,
        grid=(grid,),
        in_specs=[rowblk(16), rowblk(16), rowblk(16), full(1, 2)],
        out_specs=[rowblk(2)],
        out_shape=[jax.ShapeDtypeStruct((np_, 2), jnp.float32)],
    )(p2[0], p2[1], tab2, b2.reshape(1, 2))[0]

    return outp[:n]


# speedup vs baseline: 1.1103x; 1.1103x over previous
import jax
import jax.numpy as jnp
from jax import lax
from jax.experimental import pallas as pl
from jax.experimental.pallas import tpu as pltpu
from jax.experimental.pallas import tpu_sc as plsc


def _vgather(vec, idx16):
    return lax.gather(
        vec, idx16.reshape(16, 1),
        lax.GatherDimensionNumbers(offset_dims=(), collapsed_slice_dims=(0,),
                                   start_index_map=(0,)),
        (1,), mode=lax.GatherScatterMode.PROMISE_IN_BOUNDS)


def _lrelu_exp(t):
    return jnp.exp(jnp.where(t >= 0, t, 0.2 * t))


_NC = 2
_NS = 16
_NW = _NC * _NS
_CHUNK = 128
_ROWBLK = 1024


def _dense1_body(x_ref, w1_ref, asrc_ref, adst_ref, t1_ref, ad_ref):
    h = jnp.dot(x_ref[...], w1_ref[...], preferred_element_type=jnp.float32)
    b = h.shape[0]
    hh, cc = asrc_ref.shape
    hr = h.reshape(b, hh, cc)
    a_s = (hr * asrc_ref[...][None]).sum(-1)
    a_d = (hr * adst_ref[...][None]).sum(-1)
    z = jnp.zeros((b, 8), jnp.float32)
    t1_ref[...] = jnp.concatenate([h, a_s, z], axis=1)
    ad_ref[...] = jnp.concatenate([a_d, z], axis=1)


def _combine1_body(pa_ref, pb_ref, t1_ref, ad_ref, b1_ref, w2_ref, as2_ref,
                   ad2_ref, tab2_ref):
    acc = pa_ref[...] + pb_ref[...]
    b = acc.shape[0]
    h1 = t1_ref[:, :64]
    t = t1_ref[:, 64:72] + ad_ref[:, :8]
    wself = _lrelu_exp(t)
    num = acc[:, :64].reshape(b, 8, 8) + wself[:, :, None] * h1.reshape(b, 8, 8)
    den = acc[:, 64:72] + wself
    out1 = (num / (den[:, :, None] + 1e-16)).reshape(b, 64) + b1_ref[...]
    g = jnp.where(out1 > 0, out1, jnp.exp(jnp.minimum(out1, 0.0)) - 1.0)
    h2 = jnp.dot(g, w2_ref[...], preferred_element_type=jnp.float32)
    h2r = h2.reshape(b, as2_ref.shape[0], as2_ref.shape[1])
    s2 = (h2r * as2_ref[...][None]).sum(-1)
    d2 = (h2r * ad2_ref[...][None]).sum(-1)
    z12 = jnp.zeros((b, 12), jnp.float32)
    tab2_ref[...] = jnp.concatenate([h2, s2, d2, z12], axis=1)


def _final_body(pa_ref, pb_ref, tab2_ref, b2_ref, out_ref):
    acc = pa_ref[...] + pb_ref[...]
    h2 = tab2_ref[:, :2]
    t = tab2_ref[:, 2:3] + tab2_ref[:, 3:4]
    wself = _lrelu_exp(t)
    num = acc[:, :2] + wself * h2
    den = acc[:, 2:3] + wself
    o = num / (den + 1e-16) + b2_ref[...]
    m = jnp.max(o, axis=1, keepdims=True)
    out_ref[...] = (o - m) - jnp.log(jnp.exp(o - m).sum(axis=1, keepdims=True))


def _make_edge1_body(nch_a, nch_b):
  def _edge1_body(t1, ad1, srcg, dstg, zer, out, srcv, dstv,
                  trows0, trows1, adrows0, adrows1, sbuf0, sbuf1,
                  gt0, gt1, ga0, ga1, ss0, ss1, acc):
    cid = lax.axis_index("c")
    sid = lax.axis_index("s")
    wid = sid * _NC + cid
    if nch_a == nch_b:
        nch_w, npair = nch_a, nch_a // 2
    else:
        nch_w = jnp.where(cid == 0, nch_a, nch_b)
        npair = jnp.where(cid == 0, nch_a // 2, nch_b // 2)
    rows_per = acc.shape[0] // _NS
    trows = [trows0, trows1]
    adrows = [adrows0, adrows1]
    sbuf = [sbuf0, sbuf1]
    gt = [gt0, gt1]
    ga = [ga0, ga1]
    ss = [ss0, ss1]

    pltpu.sync_copy(srcg.at[wid], srcv)
    pltpu.sync_copy(dstg.at[wid], dstv)
    pltpu.sync_copy(zer.at[pl.ds(sid * rows_per, rows_per)],
                    acc.at[pl.ds(sid * rows_per, rows_per)])
    plsc.subcore_barrier()

    lane = lax.iota(jnp.int32, 16)
    lane8 = lane < 8
    hi8 = lane >> 3
    widx = [hi8 + 2 * jj for jj in range(4)]

    for b in range(2):
        pltpu.async_copy(t1.at[srcv.at[b]], trows[b], gt[b])
        pltpu.async_copy(ad1.at[dstv.at[b]], adrows[b], ga[b])

    def _pair(jo, carry):
        for b in range(2):
            j = 2 * jo + b

            @pl.when(jo >= 1)
            def _():
                pltpu.make_async_copy(sbuf[b], acc.at[dstv.at[j]], ss[b]).wait()

            pltpu.make_async_copy(t1.at[srcv.at[j]], trows[b], gt[b]).wait()
            pltpu.make_async_copy(ad1.at[dstv.at[j]], adrows[b], ga[b]).wait()

            @plsc.parallel_loop(0, _CHUNK, step=1, unroll=4)
            def _edge(e):
                as_v = trows[b][e, pl.ds(64, 16)]
                ad_v = adrows[b][e, pl.ds(0, 16)]
                w = _lrelu_exp(as_v + ad_v)
                sbuf[b][e, pl.ds(64, 16)] = jnp.where(lane8, w, 0.0)
                for jj in range(4):
                    wexp = _vgather(w, widx[jj])
                    sbuf[b][e, pl.ds(jj * 16, 16)] = (
                        trows[b][e, pl.ds(jj * 16, 16)] * wexp)

            pltpu.async_copy(sbuf[b], acc.at[dstv.at[j]], ss[b], add=True)

            @pl.when(jo < npair - 1)
            def _():
                pltpu.async_copy(t1.at[srcv.at[j + 2]], trows[b], gt[b])
                pltpu.async_copy(ad1.at[dstv.at[j + 2]], adrows[b], ga[b])
        return carry
    lax.fori_loop(0, npair, _pair, 0)

    for b in range(2):
        pltpu.make_async_copy(sbuf[b], acc.at[dstv.at[nch_w - 2 + b]],
                              ss[b]).wait()

    plsc.subcore_barrier()
    pltpu.sync_copy(acc.at[pl.ds(sid * rows_per, rows_per)],
                    out.at[cid].at[pl.ds(sid * rows_per, rows_per)])
  return _edge1_body


def _make_edge2_body(nch_a, nch_b):
  def _edge2_body(tab2h, srcg, dstg, zer, out, srcv, dstv,
                  srows0, srows1, drows0, drows1, s20, s21,
                  gt0, gt1, ga0, ga1, ss0, ss1, acc):
    cid = lax.axis_index("c")
    sid = lax.axis_index("s")
    wid = sid * _NC + cid
    if nch_a == nch_b:
        nch_w, npair = nch_a, nch_a // 2
    else:
        nch_w = jnp.where(cid == 0, nch_a, nch_b)
        npair = jnp.where(cid == 0, nch_a // 2, nch_b // 2)
    rows_per = acc.shape[0] // _NS
    srows = [srows0, srows1]
    drows = [drows0, drows1]
    s2 = [s20, s21]
    gt = [gt0, gt1]
    ga = [ga0, ga1]
    ss = [ss0, ss1]

    pltpu.sync_copy(srcg.at[wid], srcv)
    pltpu.sync_copy(dstg.at[wid], dstv)
    pltpu.sync_copy(zer.at[pl.ds(sid * rows_per, rows_per)],
                    acc.at[pl.ds(sid * rows_per, rows_per)])
    plsc.subcore_barrier()

    lane = lax.iota(jnp.int32, 16)
    l0 = lane == 0
    l1 = lane == 1
    l2 = lane == 2
    f0 = lane * 0
    f1 = f0 + 1
    f2 = f0 + 2
    f3 = f0 + 3

    for b in range(2):
        pltpu.async_copy(tab2h.at[srcv.at[b]], srows[b], gt[b])
        pltpu.async_copy(tab2h.at[dstv.at[b]], drows[b], ga[b])

    def _pair(jo, carry):
        for b in range(2):
            j = 2 * jo + b

            @pl.when(jo >= 1)
            def _():
                pltpu.make_async_copy(s2[b], acc.at[dstv.at[j]], ss[b]).wait()

            pltpu.make_async_copy(tab2h.at[srcv.at[j]], srows[b], gt[b]).wait()
            pltpu.make_async_copy(tab2h.at[dstv.at[j]], drows[b], ga[b]).wait()

            @plsc.parallel_loop(0, _CHUNK, step=1, unroll=4)
            def _edge(e):
                sr = srows[b][e, pl.ds(0, 16)]
                dr = drows[b][e, pl.ds(0, 16)]
                t = _vgather(sr, f2) + _vgather(dr, f3)
                w = _lrelu_exp(t)
                h0 = _vgather(sr, f0)
                h1 = _vgather(sr, f1)
                s2[b][e, pl.ds(0, 16)] = w * jnp.where(
                    l0, h0, jnp.where(l1, h1, jnp.where(l2, 1.0, 0.0)))

            pltpu.async_copy(s2[b], acc.at[dstv.at[j]], ss[b], add=True)

            @pl.when(jo < npair - 1)
            def _():
                pltpu.async_copy(tab2h.at[srcv.at[j + 2]], srows[b], gt[b])
                pltpu.async_copy(tab2h.at[dstv.at[j + 2]], drows[b], ga[b])
        return carry
    lax.fori_loop(0, npair, _pair, 0)

    for b in range(2):
        pltpu.make_async_copy(s2[b], acc.at[dstv.at[nch_w - 2 + b]],
                              ss[b]).wait()

    plsc.subcore_barrier()
    pltpu.sync_copy(acc.at[pl.ds(sid * rows_per, rows_per)],
                    out.at[cid].at[pl.ds(sid * rows_per, rows_per)])
  return _edge2_body


def kernel(x, edge_index, W1, att_src1, att_dst1, b1, W2, att_src2, att_dst2,
           b2):
    n, d = x.shape
    e = edge_index.shape[1]
    np_ = ((n + _ROWBLK - 1) // _ROWBLK) * _ROWBLK
    nch = (e + _NW * _CHUNK - 1) // (_NW * _CHUNK)
    nch = ((nch + 1) // 2) * 2

    nch_a = nch_b = nchm = nch
    epad = _NW * nch * _CHUNK
    src = jnp.concatenate(
        [edge_index[0], jnp.full((epad - e,), n, jnp.int32)]).reshape(
            _NW, nch, _CHUNK)
    dst = jnp.concatenate(
        [edge_index[1], jnp.full((epad - e,), n, jnp.int32)]).reshape(
            _NW, nch, _CHUNK)
    xp = jnp.pad(x, ((0, np_ - n), (0, 0)))
    zeros80 = jnp.zeros((np_, 80), jnp.float32)
    zeros16 = jnp.zeros((np_, 16), jnp.float32)

    grid = np_ // _ROWBLK
    full = lambda *shape: pl.BlockSpec(shape, lambda i: (0,) * len(shape))
    rowblk = lambda *rest: pl.BlockSpec((_ROWBLK,) + rest,
                                        lambda i: (i,) + (0,) * len(rest))

    t1, ad1 = pl.pallas_call(
        _dense1_body,
        grid=(grid,),
        in_specs=[rowblk(d), full(d, 64), full(8, 8), full(8, 8)],
        out_specs=[rowblk(80), rowblk(16)],
        out_shape=[jax.ShapeDtypeStruct((np_, 80), jnp.float32),
                   jax.ShapeDtypeStruct((np_, 16), jnp.float32)],
    )(xp, W1, att_src1, att_dst1)

    mesh = plsc.VectorSubcoreMesh(core_axis_name="c", subcore_axis_name="s")
    scparams = pltpu.CompilerParams(use_tc_tiling_on_sc=False)
    p1 = pl.kernel(
        _make_edge1_body(nch_a, nch_b),
        out_type=jax.ShapeDtypeStruct((_NC, np_, 80), jnp.float32),
        mesh=mesh,
        compiler_params=scparams,
        scratch_types=[
            pltpu.VMEM((nchm, _CHUNK), jnp.int32),
            pltpu.VMEM((nchm, _CHUNK), jnp.int32),
            pltpu.VMEM((_CHUNK, 80), jnp.float32),
            pltpu.VMEM((_CHUNK, 80), jnp.float32),
            pltpu.VMEM((_CHUNK, 16), jnp.float32),
            pltpu.VMEM((_CHUNK, 16), jnp.float32),
            pltpu.VMEM((_CHUNK, 80), jnp.float32),
            pltpu.VMEM((_CHUNK, 80), jnp.float32),
            pltpu.SemaphoreType.DMA,
            pltpu.SemaphoreType.DMA,
            pltpu.SemaphoreType.DMA,
            pltpu.SemaphoreType.DMA,
            pltpu.SemaphoreType.DMA,
            pltpu.SemaphoreType.DMA,
            pltpu.VMEM_SHARED((np_, 80), jnp.float32),
        ],
    )(t1, ad1, src, dst, zeros80)

    tab2 = pl.pallas_call(
        _combine1_body,
        grid=(grid,),
        in_specs=[rowblk(80), rowblk(80), rowblk(80), rowblk(16), full(1, 64),
                  full(64, 2), full(1, 2), full(1, 2)],
        out_specs=[rowblk(16)],
        out_shape=[jax.ShapeDtypeStruct((np_, 16), jnp.float32)],
    )(p1[0], p1[1], t1, ad1, b1.reshape(1, 64), W2, att_src2, att_dst2)[0]

    p2 = pl.kernel(
        _make_edge2_body(nch_a, nch_b),
        out_type=jax.ShapeDtypeStruct((_NC, np_, 16), jnp.float32),
        mesh=mesh,
        compiler_params=scparams,
        scratch_types=[
            pltpu.VMEM((nchm, _CHUNK), jnp.int32),
            pltpu.VMEM((nchm, _CHUNK), jnp.int32),
            pltpu.VMEM((_CHUNK, 16), jnp.float32),
            pltpu.VMEM((_CHUNK, 16), jnp.float32),
            pltpu.VMEM((_CHUNK, 16), jnp.float32),
            pltpu.VMEM((_CHUNK, 16), jnp.float32),
            pltpu.VMEM((_CHUNK, 16), jnp.float32),
            pltpu.VMEM((_CHUNK, 16), jnp.float32),
            pltpu.SemaphoreType.DMA,
            pltpu.SemaphoreType.DMA,
            pltpu.SemaphoreType.DMA,
            pltpu.SemaphoreType.DMA,
            pltpu.SemaphoreType.DMA,
            pltpu.SemaphoreType.DMA,
            pltpu.VMEM_SHARED((np_, 16), jnp.float32),
        ],
    )(tab2, src, dst, zeros16)

    outp = pl.pallas_call(
        _final_body,
        grid=(grid,),
        in_specs=[rowblk(16), rowblk(16), rowblk(16), full(1, 2)],
        out_specs=[rowblk(2)],
        out_shape=[jax.ShapeDtypeStruct((np_, 2), jnp.float32)],
    )(p2[0], p2[1], tab2, b2.reshape(1, 2))[0]

    return outp[:n]
